# baseline (speedup 1.0000x reference)
import jax
import jax.numpy as jnp
from jax import lax
from jax.experimental import pallas as pl
from jax.experimental.pallas import tpu as pltpu
from jax.experimental.pallas import tpu_sc as plsc

B = 16384
L = 200
NC = 2
NS = 16
NW = NC * NS
COLS_PER_W = B // NW
NGROUP = COLS_PER_W // 16
LCHUNK = 8
NCHUNK = L // LCHUNK
PIPE = 4
TSTRIDE = 17


def _sc_body(x_hbm, w_hbm, out_hbm, bb, wv, accv, sem):
    wid = lax.axis_index("s") * NC + lax.axis_index("c")
    base = wid * COLS_PER_W
    pltpu.sync_copy(w_hbm, wv)

    tbase = lax.iota(jnp.int32, 16) * TSTRIDE

    def issue(i):
        pltpu.async_copy(
            x_hbm.at[pl.ds(i * LCHUNK, LCHUNK), pl.ds(base, COLS_PER_W)],
            bb.at[pl.ds(i * LCHUNK, LCHUNK), :], sem)

    def zero_group(g, _):
        accv[pl.ds(g * 16, 16)] = jnp.zeros((16,), jnp.float32)
        return 0

    lax.fori_loop(0, NGROUP, zero_group, 0)

    for i in range(PIPE):
        issue(i)

    def chunk_body(i, _):
        pltpu.make_async_copy(
            x_hbm.at[pl.ds(0, LCHUNK), pl.ds(base, COLS_PER_W)],
            bb.at[pl.ds(0, LCHUNK), :], sem).wait()

        @pl.when(i + PIPE < NCHUNK)
        def _():
            issue(i + PIPE)

        for g in range(NGROUP):
            gl = g * 16
            gath = []
            for l in range(LCHUNK):
                v = bb[i * LCHUNK + l, pl.ds(gl, 16)]
                gath.append(plsc.load_gather(wv, [v + tbase]))
            while len(gath) > 1:
                gath = [a + b for a, b in zip(gath[::2], gath[1::2])]
            plsc.addupdate(accv.at[pl.ds(gl, 16)], gath[0])
        return 0

    lax.fori_loop(0, NCHUNK, chunk_body, 0)
    pltpu.sync_copy(accv, out_hbm.at[pl.ds(base, COLS_PER_W)])


@jax.jit
def _sc_call(x_t, w_rep):
    mesh = plsc.VectorSubcoreMesh(core_axis_name="c", subcore_axis_name="s")
    f = pl.kernel(
        _sc_body,
        out_type=jax.ShapeDtypeStruct((B,), jnp.float32),
        mesh=mesh,
        scratch_types=[
            pltpu.VMEM((L, COLS_PER_W), jnp.int32),
            pltpu.VMEM((16 * TSTRIDE,), jnp.float32),
            pltpu.VMEM((COLS_PER_W,), jnp.float32),
            pltpu.SemaphoreType.DMA,
        ],
        compiler_params=pltpu.CompilerParams(
            use_tc_tiling_on_sc=True, needs_layout_passes=False),
    )
    return f(x_t, w_rep)


def kernel(x, emb_weight):
    w_pad = jnp.concatenate(
        [emb_weight[:, 0], jnp.zeros((TSTRIDE - 5,), jnp.float32)])
    w_rep = jnp.tile(w_pad, 16)
    out = _sc_call(x.T, w_rep)
    return out.reshape(B, 1)


# speedup vs baseline: 1.0093x; 1.0093x over previous
import jax
import jax.numpy as jnp
from jax import lax
from jax.experimental import pallas as pl
from jax.experimental.pallas import tpu as pltpu
from jax.experimental.pallas import tpu_sc as plsc

B = 16384
L = 200
NC = 2
NS = 16
NW = NC * NS
COLS_PER_W = B // NW
NGROUP = COLS_PER_W // 16
LCHUNK = 8
NCHUNK = L // LCHUNK
PIPE = 4


def _sc_body(x_hbm, w_hbm, out_hbm, bb, wv, accv, sem):
    wid = lax.axis_index("s") * NC + lax.axis_index("c")
    base = wid * COLS_PER_W
    pltpu.sync_copy(w_hbm, wv)

    wvec = wv[...]
    ws = [jnp.broadcast_to(wvec[k], (16,)) for k in range(5)]

    def lookup(v):
        val = jnp.where(v == 1, ws[1], ws[0])
        val = jnp.where(v == 2, ws[2], val)
        val = jnp.where(v == 3, ws[3], val)
        return jnp.where(v == 4, ws[4], val)

    def issue(i):
        pltpu.async_copy(
            x_hbm.at[pl.ds(i * LCHUNK, LCHUNK), pl.ds(base, COLS_PER_W)],
            bb.at[pl.ds(i * LCHUNK, LCHUNK), :], sem)

    def zero_group(g, _):
        accv[pl.ds(g * 16, 16)] = jnp.zeros((16,), jnp.float32)
        return 0

    lax.fori_loop(0, NGROUP, zero_group, 0)

    for i in range(PIPE):
        issue(i)

    def chunk_body(i, _):
        pltpu.make_async_copy(
            x_hbm.at[pl.ds(0, LCHUNK), pl.ds(base, COLS_PER_W)],
            bb.at[pl.ds(0, LCHUNK), :], sem).wait()

        @pl.when(i + PIPE < NCHUNK)
        def _():
            issue(i + PIPE)

        for g in range(NGROUP):
            gl = g * 16
            vals = []
            for l in range(LCHUNK):
                v = bb[i * LCHUNK + l, pl.ds(gl, 16)]
                vals.append(lookup(v))
            while len(vals) > 1:
                vals = [a + b for a, b in zip(vals[::2], vals[1::2])]
            plsc.addupdate(accv.at[pl.ds(gl, 16)], vals[0])
        return 0

    lax.fori_loop(0, NCHUNK, chunk_body, 0)
    pltpu.sync_copy(accv, out_hbm.at[pl.ds(base, COLS_PER_W)])


@jax.jit
def _sc_call(x_t, w5):
    mesh = plsc.VectorSubcoreMesh(core_axis_name="c", subcore_axis_name="s")
    f = pl.kernel(
        _sc_body,
        out_type=jax.ShapeDtypeStruct((B,), jnp.float32),
        mesh=mesh,
        scratch_types=[
            pltpu.VMEM((L, COLS_PER_W), jnp.int32),
            pltpu.VMEM((16,), jnp.float32),
            pltpu.VMEM((COLS_PER_W,), jnp.float32),
            pltpu.SemaphoreType.DMA,
        ],
        compiler_params=pltpu.CompilerParams(
            use_tc_tiling_on_sc=True, needs_layout_passes=False),
    )
    return f(x_t, w5)


def kernel(x, emb_weight):
    w16 = jnp.zeros((16,), jnp.float32).at[:5].set(emb_weight[:, 0])
    out = _sc_call(x.T, w16)
    return out.reshape(B, 1)


# speedup vs baseline: 1.0190x; 1.0096x over previous
import jax
import jax.numpy as jnp
from jax import lax
from jax.experimental import pallas as pl
from jax.experimental.pallas import tpu as pltpu
from jax.experimental.pallas import tpu_sc as plsc

B = 16384
L = 200
NC = 2
NS = 16
NW = NC * NS
COLS_PER_W = B // NW
NGROUP = COLS_PER_W // 16
LCHUNK = 8
NCHUNK = L // LCHUNK
PIPE = 6


def _sc_body(x_hbm, w_hbm, out_hbm, bb, wv, accv, sem):
    wid = lax.axis_index("s") * NC + lax.axis_index("c")
    base = wid * COLS_PER_W
    pltpu.sync_copy(w_hbm, wv)

    wvec = wv[...]
    ws = [jnp.broadcast_to(wvec[k], (16,)) for k in range(5)]

    def lookup(v):
        val = jnp.where(v == 1, ws[1], ws[0])
        val = jnp.where(v == 2, ws[2], val)
        val = jnp.where(v == 3, ws[3], val)
        return jnp.where(v == 4, ws[4], val)

    def issue(i):
        pltpu.async_copy(
            x_hbm.at[pl.ds(i * LCHUNK, LCHUNK), pl.ds(base, COLS_PER_W)],
            bb.at[pl.ds(i * LCHUNK, LCHUNK), :], sem)

    def zero_group(g, _):
        accv[pl.ds(g * 16, 16)] = jnp.zeros((16,), jnp.float32)
        return 0

    lax.fori_loop(0, NGROUP, zero_group, 0)

    for i in range(PIPE):
        issue(i)

    def chunk_body(i, _):
        pltpu.make_async_copy(
            x_hbm.at[pl.ds(0, LCHUNK), pl.ds(base, COLS_PER_W)],
            bb.at[pl.ds(0, LCHUNK), :], sem).wait()

        @pl.when(i + PIPE < NCHUNK)
        def _():
            issue(i + PIPE)

        def group_body(g, _):
            gl = g * 16
            vals = []
            for l in range(LCHUNK):
                v = bb[i * LCHUNK + l, pl.ds(gl, 16)]
                vals.append(lookup(v))
            while len(vals) > 1:
                vals = [a + b for a, b in zip(vals[::2], vals[1::2])]
            plsc.addupdate(accv.at[pl.ds(gl, 16)], vals[0])
            return 0

        lax.fori_loop(0, NGROUP, group_body, 0)
        return 0

    lax.fori_loop(0, NCHUNK, chunk_body, 0)
    pltpu.sync_copy(accv, out_hbm.at[pl.ds(base, COLS_PER_W)])


@jax.jit
def _sc_call(x_t, w5):
    mesh = plsc.VectorSubcoreMesh(core_axis_name="c", subcore_axis_name="s")
    f = pl.kernel(
        _sc_body,
        out_type=jax.ShapeDtypeStruct((B,), jnp.float32),
        mesh=mesh,
        scratch_types=[
            pltpu.VMEM((L, COLS_PER_W), jnp.int32),
            pltpu.VMEM((16,), jnp.float32),
            pltpu.VMEM((COLS_PER_W,), jnp.float32),
            pltpu.SemaphoreType.DMA,
        ],
        compiler_params=pltpu.CompilerParams(
            use_tc_tiling_on_sc=True, needs_layout_passes=False),
    )
    return f(x_t, w5)


def kernel(x, emb_weight):
    w16 = jnp.zeros((16,), jnp.float32).at[:5].set(emb_weight[:, 0])
    out = _sc_call(x.T, w16)
    return out.reshape(B, 1)


# speedup vs baseline: 1.1982x; 1.1759x over previous
import jax
import jax.numpy as jnp
from jax import lax
from jax.experimental import pallas as pl
from jax.experimental.pallas import tpu as pltpu
from jax.experimental.pallas import tpu_sc as plsc

B = 16384
L = 200
NC = 2
NS = 16
NW = NC * NS
COLS_PER_W = B // NW
NGROUP = COLS_PER_W // 16
LCHUNK = 40
NCHUNK = L // LCHUNK
PIPE = 2


def _sc_body(x_hbm, w_hbm, out_hbm, bb, wv, accv, sem):
    wid = lax.axis_index("s") * NC + lax.axis_index("c")
    base = wid * COLS_PER_W
    pltpu.sync_copy(w_hbm, wv)

    wvec = wv[...]
    ws = [jnp.broadcast_to(wvec[k], (16,)) for k in range(5)]

    def lookup(v):
        val = jnp.where(v == 1, ws[1], ws[0])
        val = jnp.where(v == 2, ws[2], val)
        val = jnp.where(v == 3, ws[3], val)
        return jnp.where(v == 4, ws[4], val)

    def issue(i):
        pltpu.async_copy(
            x_hbm.at[pl.ds(i * LCHUNK, LCHUNK), pl.ds(base, COLS_PER_W)],
            bb.at[pl.ds(i * LCHUNK, LCHUNK), :], sem)

    def zero_group(g, _):
        accv[pl.ds(g * 16, 16)] = jnp.zeros((16,), jnp.float32)
        return 0

    lax.fori_loop(0, NGROUP, zero_group, 0)

    for i in range(PIPE):
        issue(i)

    def chunk_body(i, _):
        pltpu.make_async_copy(
            x_hbm.at[pl.ds(0, LCHUNK), pl.ds(base, COLS_PER_W)],
            bb.at[pl.ds(0, LCHUNK), :], sem).wait()

        @pl.when(i + PIPE < NCHUNK)
        def _():
            issue(i + PIPE)

        def group_body(g, _):
            gl = g * 16
            vals = []
            for l in range(LCHUNK):
                v = bb[i * LCHUNK + l, pl.ds(gl, 16)]
                vals.append(lookup(v))
            while len(vals) > 1:
                vals = [a + b for a, b in zip(vals[::2], vals[1::2])]
            plsc.addupdate(accv.at[pl.ds(gl, 16)], vals[0])
            return 0

        lax.fori_loop(0, NGROUP, group_body, 0)
        return 0

    lax.fori_loop(0, NCHUNK, chunk_body, 0)
    pltpu.sync_copy(accv, out_hbm.at[pl.ds(base, COLS_PER_W)])


@jax.jit
def _sc_call(x_t, w5):
    mesh = plsc.VectorSubcoreMesh(core_axis_name="c", subcore_axis_name="s")
    f = pl.kernel(
        _sc_body,
        out_type=jax.ShapeDtypeStruct((B,), jnp.float32),
        mesh=mesh,
        scratch_types=[
            pltpu.VMEM((L, COLS_PER_W), jnp.int32),
            pltpu.VMEM((16,), jnp.float32),
            pltpu.VMEM((COLS_PER_W,), jnp.float32),
            pltpu.SemaphoreType.DMA,
        ],
        compiler_params=pltpu.CompilerParams(
            use_tc_tiling_on_sc=True, needs_layout_passes=False),
    )
    return f(x_t, w5)


def kernel(x, emb_weight):
    w16 = jnp.zeros((16,), jnp.float32).at[:5].set(emb_weight[:, 0])
    out = _sc_call(x.T, w16)
    return out.reshape(B, 1)
